# baseline (speedup 1.0000x reference)
import jax
import jax.numpy as jnp
from jax.experimental import pallas as pl
from jax.experimental.pallas import tpu as pltpu

N = 20000
B = 8
M = 50
C = 80
NB = 16
TOPK = 9
IMG = 1024.0
GIOU_W = 1.0

NP = 20480
T = 5000
NT = N // T
HI = jax.lax.Precision.HIGHEST


def _match_kernel(anchT_ref, gt_ref, lab_ref, out_ref):
    ax0 = anchT_ref[0:1, :]
    ay0 = anchT_ref[1:2, :]
    ax1 = anchT_ref[2:3, :]
    ay1 = anchT_ref[3:4, :]
    gt = gt_ref[0]
    gx0 = gt[:, 0:1]
    gy0 = gt[:, 1:2]
    gx1 = gt[:, 2:3]
    gy1 = gt[:, 3:4]

    iw = jnp.clip(jnp.minimum(ax1, gx1) - jnp.maximum(ax0, gx0), 0.0, None)
    ih = jnp.clip(jnp.minimum(ay1, gy1) - jnp.maximum(ay0, gy0), 0.0, None)
    inter = iw * ih
    area_a = (ax1 - ax0) * (ay1 - ay0)
    area_g = (gx1 - gx0) * (gy1 - gy0)
    iou = inter / (area_a + area_g - inter + 1e-7)

    acx = (ax0 + ax1) * 0.5
    acy = (ay0 + ay1) * 0.5
    gcx = (gx0 + gx1) * 0.5
    gcy = (gy0 + gy1) * 0.5
    dx = acx - gcx
    dy = acy - gcy
    dist2 = dx * dx + dy * dy

    lane = jax.lax.broadcasted_iota(jnp.int32, (M, NP), 1)
    big_i = jnp.int32(NP + 1)
    inf = jnp.float32(jnp.inf)

    d = dist2
    vals = []
    for _ in range(TOPK):
        mn = jnp.min(d, axis=1, keepdims=True)
        ismin = d == mn
        first = jnp.min(jnp.where(ismin, lane, big_i), axis=1, keepdims=True)
        sel = lane == first
        vals.append(jnp.sum(jnp.where(sel, iou, 0.0), axis=1, keepdims=True))
        d = jnp.where(sel, inf, d)
    mean = vals[0]
    for v in vals[1:]:
        mean = mean + v
    mean = mean / TOPK
    ss = (vals[0] - mean) ** 2
    for v in vals[1:]:
        ss = ss + (v - mean) ** 2
    thr = mean + jnp.sqrt(ss / (TOPK - 1))

    inside = (acx >= gx0) & (acx <= gx1) & (acy >= gy0) & (acy <= gy1)
    pos = (iou >= thr) & inside

    m_idx = jax.lax.broadcasted_iota(jnp.int32, (M, NP), 0)
    matched = jnp.max(jnp.where(pos, m_idx, -1), axis=0, keepdims=True)
    hit = m_idx == matched
    miou = jnp.sum(jnp.where(hit, iou, 0.0), axis=0, keepdims=True)
    labf = lab_ref[0][:, 0:1]
    cls_t = jnp.sum(jnp.where(hit, labf, 0.0), axis=0, keepdims=True)
    tx0 = jnp.sum(jnp.where(hit, gx0, 0.0), axis=0, keepdims=True)
    ty0 = jnp.sum(jnp.where(hit, gy0, 0.0), axis=0, keepdims=True)
    tx1 = jnp.sum(jnp.where(hit, gx1, 0.0), axis=0, keepdims=True)
    ty1 = jnp.sum(jnp.where(hit, gy1, 0.0), axis=0, keepdims=True)
    posf = (matched >= 0).astype(jnp.float32)

    out_ref[0, 0:1, :] = cls_t
    out_ref[0, 1:2, :] = miou
    out_ref[0, 2:3, :] = tx0
    out_ref[0, 3:4, :] = ty0
    out_ref[0, 4:5, :] = tx1
    out_ref[0, 5:6, :] = ty1
    out_ref[0, 6:7, :] = posf
    out_ref[0, 7:8, :] = jnp.zeros((1, NP), jnp.float32)


def _loss_kernel(cls_ref, reg_ref, anch_ref, tgt_ref, acc_ref):
    b = pl.program_id(0)
    n = pl.program_id(1)

    @pl.when((b == 0) & (n == 0))
    def _init():
        acc_ref[...] = jnp.zeros((4, 128), jnp.float32)

    tgt = tgt_ref[0]
    cls_t = tgt[:, 0:1]
    iou_t = tgt[:, 1:2]
    tx0 = tgt[:, 2:3]
    ty0 = tgt[:, 3:4]
    tx1 = tgt[:, 4:5]
    ty1 = tgt[:, 5:6]
    posf = tgt[:, 6:7]
    npos_part = jnp.sum(posf)

    x = cls_ref[0]
    e = jnp.exp(x)
    p = e + 1.0
    s1 = 1.0 / p
    bce = jnp.log(p)
    tneg = s1 * (1.0 + iou_t * (e - 1.0))
    qfl_neg = jnp.sum(tneg * tneg * bce)
    lane_c = jax.lax.broadcasted_iota(jnp.int32, (T, C), 1)
    ohm = lane_c == cls_t.astype(jnp.int32)
    xm = jnp.sum(jnp.where(ohm, x, 0.0), axis=1, keepdims=True)
    sigm = jax.nn.sigmoid(xm)
    s1m = 1.0 - sigm
    mxm = jnp.maximum(xm, 0.0)
    lgm = jnp.log(jnp.maximum(sigm, s1m))
    bce_pos = mxm - xm - lgm
    bce_negm = mxm - lgm
    tpos = iou_t * s1m + (1.0 - iou_t) * sigm
    tnegm = iou_t * sigm + (1.0 - iou_t) * s1m
    qfl_corr = jnp.sum(tpos * tpos * bce_pos - tnegm * tnegm * bce_negm)
    qfl_part = qfl_neg + qfl_corr

    x64 = reg_ref[0]
    e = jnp.exp(x64)
    r64 = jax.lax.broadcasted_iota(jnp.int32, (64, 8), 0)
    c8 = jax.lax.broadcasted_iota(jnp.int32, (64, 8), 1)
    ind = (r64 // NB) == (c8 % 4)
    g2 = jnp.where(ind,
                   jnp.where(c8 < 4, 1.0, (r64 % NB).astype(jnp.float32)),
                   0.0)
    s8 = jax.lax.dot_general(e, g2, (((1,), (0,)), ((), ())), precision=HI)
    s4 = s8[:, 0:4]
    w4 = s8[:, 4:8]
    logz4 = jnp.log(s4)
    logzsum = jnp.sum(logz4, axis=1, keepdims=True)
    deltas4 = (w4 / s4) / (NB - 1) - 0.5

    tgt4 = tgt[:, 2:6]
    scaled = jnp.clip(tgt4 / IMG, 0.0, 1.0) * (NB - 1)
    r4 = jax.lax.broadcasted_iota(jnp.int32, (4, 64), 0)
    c64 = jax.lax.broadcasted_iota(jnp.int32, (4, 64), 1)
    gb = ((c64 // NB) == r4).astype(jnp.float32)
    scaledb = jax.lax.dot_general(scaled, gb, (((1,), (0,)), ((), ())),
                                  precision=HI)
    lmf = (jax.lax.broadcasted_iota(jnp.int32, (T, 64), 1) % NB).astype(
        jnp.float32)
    contrib = jnp.maximum(1.0 - jnp.abs(scaledb - lmf), 0.0)
    posfb = jnp.broadcast_to(posf, (T, 64))
    dfl_part = jnp.sum(posf * logzsum) - jnp.sum(posfb * contrib * x64)

    ax0 = anch_ref[:, 0:1]
    ay0 = anch_ref[:, 1:2]
    ax1 = anch_ref[:, 2:3]
    ay1 = anch_ref[:, 3:4]
    wA = ax1 - ax0
    hA = ay1 - ay0
    cxA = ax0 + 0.5 * wA
    cyA = ay0 + 0.5 * hA
    dxv = deltas4[:, 0:1]
    dyv = deltas4[:, 1:2]
    dwv = deltas4[:, 2:3]
    dhv = deltas4[:, 3:4]
    pcx = dxv * wA + cxA
    pcy = dyv * hA + cyA
    pw = jnp.exp(dwv) * wA
    ph = jnp.exp(dhv) * hA
    pb0 = pcx - 0.5 * pw
    pb1 = pcy - 0.5 * ph
    pb2 = pcx + 0.5 * pw
    pb3 = pcy + 0.5 * ph
    area_p = (pb2 - pb0) * (pb3 - pb1)
    area_t = (tx1 - tx0) * (ty1 - ty0)
    iw = jnp.clip(jnp.minimum(pb2, tx1) - jnp.maximum(pb0, tx0), 0.0, None)
    ih = jnp.clip(jnp.minimum(pb3, ty1) - jnp.maximum(pb1, ty0), 0.0, None)
    inter = iw * ih
    union = area_p + area_t - inter
    iou = inter / (union + 1e-7)
    ew = jnp.maximum(pb2, tx1) - jnp.minimum(pb0, tx0)
    eh = jnp.maximum(pb3, ty1) - jnp.minimum(pb1, ty0)
    enc = ew * eh
    giou = iou - (enc - union) / (enc + 1e-7)
    gl_part = jnp.sum((1.0 - giou) * posf)

    ones = jnp.ones((1, 128), jnp.float32)
    upd = jnp.concatenate(
        [qfl_part * ones, npos_part * ones, dfl_part * ones, gl_part * ones], axis=0)
    acc_ref[...] = acc_ref[...] + upd


def kernel(cls_out, reg_out, anchors, gt_boxes, gt_labels):
    pad = jnp.tile(jnp.array([[1e8, 1e8, 1e8 + 8.0, 1e8 + 8.0]], jnp.float32),
                   (NP - N, 1))
    anchT = jnp.concatenate([anchors, pad], axis=0).T
    labf = gt_labels.astype(jnp.float32).reshape(B, M, 1)

    tgt = pl.pallas_call(
        _match_kernel,
        grid=(B,),
        in_specs=[
            pl.BlockSpec((4, NP), lambda b: (0, 0)),
            pl.BlockSpec((1, M, 4), lambda b: (b, 0, 0)),
            pl.BlockSpec((1, M, 1), lambda b: (b, 0, 0)),
        ],
        out_specs=pl.BlockSpec((1, 8, NP), lambda b: (b, 0, 0)),
        out_shape=jax.ShapeDtypeStruct((B, 8, NP), jnp.float32),
    )(anchT, gt_boxes, labf)

    tgtT = tgt[:, :, :N].transpose(0, 2, 1)

    acc = pl.pallas_call(
        _loss_kernel,
        grid=(B, NT),
        in_specs=[
            pl.BlockSpec((1, T, C), lambda b, n: (b, n, 0)),
            pl.BlockSpec((1, T, 4 * NB), lambda b, n: (b, n, 0)),
            pl.BlockSpec((T, 4), lambda b, n: (n, 0)),
            pl.BlockSpec((1, T, 8), lambda b, n: (b, n, 0)),
        ],
        out_specs=pl.BlockSpec((4, 128), lambda b, n: (0, 0)),
        out_shape=jax.ShapeDtypeStruct((4, 128), jnp.float32),
    )(cls_out, reg_out, anchors, tgtT)

    accs = acc
    qfl_sum = accs[0, 0]
    npos = accs[1, 0]
    dfl_sum = accs[2, 0]
    gl_sum = accs[3, 0]
    qfl = qfl_sum / jnp.maximum(npos, 1.0)
    dfl = dfl_sum / jnp.maximum(npos * 4.0, 1.0)
    gl = gl_sum / jnp.maximum(npos, 1.0)
    return qfl + dfl + GIOU_W * gl


# speedup vs baseline: 1.1672x; 1.1672x over previous
import jax
import jax.numpy as jnp
from jax.experimental import pallas as pl
from jax.experimental.pallas import tpu as pltpu

N = 20000
B = 8
M = 50
C = 80
NB = 16
TOPK = 9
IMG = 1024.0
GIOU_W = 1.0

NP = 20480
T = 4000
NT = N // T
HI = jax.lax.Precision.HIGHEST


def _match_kernel(anchT_ref, gt_ref, lab_ref, out_ref):
    ax0 = anchT_ref[0:1, :]
    ay0 = anchT_ref[1:2, :]
    ax1 = anchT_ref[2:3, :]
    ay1 = anchT_ref[3:4, :]
    gt = gt_ref[0]
    gx0 = gt[:, 0:1]
    gy0 = gt[:, 1:2]
    gx1 = gt[:, 2:3]
    gy1 = gt[:, 3:4]

    iw = jnp.clip(jnp.minimum(ax1, gx1) - jnp.maximum(ax0, gx0), 0.0, None)
    ih = jnp.clip(jnp.minimum(ay1, gy1) - jnp.maximum(ay0, gy0), 0.0, None)
    inter = iw * ih
    area_a = (ax1 - ax0) * (ay1 - ay0)
    area_g = (gx1 - gx0) * (gy1 - gy0)
    iou = inter / (area_a + area_g - inter + 1e-7)

    acx = (ax0 + ax1) * 0.5
    acy = (ay0 + ay1) * 0.5
    gcx = (gx0 + gx1) * 0.5
    gcy = (gy0 + gy1) * 0.5
    dx = acx - gcx
    dy = acy - gcy
    dist2 = dx * dx + dy * dy

    lane = jax.lax.broadcasted_iota(jnp.int32, (M, NP), 1)
    big_i = jnp.int32(NP + 1)
    inf = jnp.float32(jnp.inf)

    d = dist2
    vals = []
    for _ in range(TOPK):
        mn = jnp.min(d, axis=1, keepdims=True)
        ismin = d == mn
        first = jnp.min(jnp.where(ismin, lane, big_i), axis=1, keepdims=True)
        sel = lane == first
        vals.append(jnp.sum(jnp.where(sel, iou, 0.0), axis=1, keepdims=True))
        d = jnp.where(sel, inf, d)
    mean = vals[0]
    for v in vals[1:]:
        mean = mean + v
    mean = mean / TOPK
    ss = (vals[0] - mean) ** 2
    for v in vals[1:]:
        ss = ss + (v - mean) ** 2
    thr = mean + jnp.sqrt(ss / (TOPK - 1))

    inside = (acx >= gx0) & (acx <= gx1) & (acy >= gy0) & (acy <= gy1)
    pos = (iou >= thr) & inside

    m_idx = jax.lax.broadcasted_iota(jnp.int32, (M, NP), 0)
    matched = jnp.max(jnp.where(pos, m_idx, -1), axis=0, keepdims=True)
    hit = m_idx == matched
    miou = jnp.sum(jnp.where(hit, iou, 0.0), axis=0, keepdims=True)
    labf = lab_ref[0][:, 0:1]
    cls_t = jnp.sum(jnp.where(hit, labf, 0.0), axis=0, keepdims=True)
    tx0 = jnp.sum(jnp.where(hit, gx0, 0.0), axis=0, keepdims=True)
    ty0 = jnp.sum(jnp.where(hit, gy0, 0.0), axis=0, keepdims=True)
    tx1 = jnp.sum(jnp.where(hit, gx1, 0.0), axis=0, keepdims=True)
    ty1 = jnp.sum(jnp.where(hit, gy1, 0.0), axis=0, keepdims=True)
    posf = (matched >= 0).astype(jnp.float32)

    out_ref[0, 0:1, :] = cls_t
    out_ref[0, 1:2, :] = miou
    out_ref[0, 2:3, :] = tx0
    out_ref[0, 3:4, :] = ty0
    out_ref[0, 4:5, :] = tx1
    out_ref[0, 5:6, :] = ty1
    out_ref[0, 6:7, :] = posf
    out_ref[0, 7:8, :] = jnp.zeros((1, NP), jnp.float32)


def _loss_kernel(cls_ref, reg_ref, anch_ref, tgt_ref, acc_ref):
    b = pl.program_id(0)
    n = pl.program_id(1)

    @pl.when((b == 0) & (n == 0))
    def _init():
        acc_ref[...] = jnp.zeros((4, 128), jnp.float32)

    tgt = tgt_ref[0]
    cls_t = tgt[:, 0:1]
    iou_t = tgt[:, 1:2]
    tx0 = tgt[:, 2:3]
    ty0 = tgt[:, 3:4]
    tx1 = tgt[:, 4:5]
    ty1 = tgt[:, 5:6]
    posf = tgt[:, 6:7]
    npos_part = jnp.sum(posf)

    x = cls_ref[0]
    e = jnp.exp(x)
    p = e + 1.0
    s1 = 1.0 / p
    bce = jnp.log(p)
    tneg = s1 * (1.0 + iou_t * (e - 1.0))
    qfl_neg = jnp.sum(tneg * tneg * bce)
    lane_c = jax.lax.broadcasted_iota(jnp.int32, (T, C), 1)
    ohm = lane_c == cls_t.astype(jnp.int32)
    xm = jnp.sum(jnp.where(ohm, x, 0.0), axis=1, keepdims=True)
    sigm = jax.nn.sigmoid(xm)
    s1m = 1.0 - sigm
    mxm = jnp.maximum(xm, 0.0)
    lgm = jnp.log(jnp.maximum(sigm, s1m))
    bce_pos = mxm - xm - lgm
    bce_negm = mxm - lgm
    tpos = iou_t * s1m + (1.0 - iou_t) * sigm
    tnegm = iou_t * sigm + (1.0 - iou_t) * s1m
    qfl_corr = jnp.sum(tpos * tpos * bce_pos - tnegm * tnegm * bce_negm)
    qfl_part = qfl_neg + qfl_corr

    x64 = reg_ref[0]
    e = jnp.exp(x64)
    r64 = jax.lax.broadcasted_iota(jnp.int32, (64, 8), 0)
    c8 = jax.lax.broadcasted_iota(jnp.int32, (64, 8), 1)
    ind = (r64 // NB) == (c8 % 4)
    g2 = jnp.where(ind,
                   jnp.where(c8 < 4, 1.0, (r64 % NB).astype(jnp.float32)),
                   0.0)
    s8 = jax.lax.dot_general(e, g2, (((1,), (0,)), ((), ())), precision=HI)
    s4 = s8[:, 0:4]
    w4 = s8[:, 4:8]
    logz4 = jnp.log(s4)
    logzsum = jnp.sum(logz4, axis=1, keepdims=True)
    deltas4 = (w4 / s4) / (NB - 1) - 0.5

    tgt4 = tgt[:, 2:6]
    scaled = jnp.clip(tgt4 / IMG, 0.0, 1.0) * (NB - 1)
    r4 = jax.lax.broadcasted_iota(jnp.int32, (4, 64), 0)
    c64 = jax.lax.broadcasted_iota(jnp.int32, (4, 64), 1)
    gb = ((c64 // NB) == r4).astype(jnp.float32)
    scaledb = jax.lax.dot_general(scaled, gb, (((1,), (0,)), ((), ())),
                                  precision=HI)
    lmf = (jax.lax.broadcasted_iota(jnp.int32, (T, 64), 1) % NB).astype(
        jnp.float32)
    contrib = jnp.maximum(1.0 - jnp.abs(scaledb - lmf), 0.0)
    posfb = jnp.broadcast_to(posf, (T, 64))
    dfl_part = jnp.sum(posf * logzsum) - jnp.sum(posfb * contrib * x64)

    ax0 = anch_ref[:, 0:1]
    ay0 = anch_ref[:, 1:2]
    ax1 = anch_ref[:, 2:3]
    ay1 = anch_ref[:, 3:4]
    wA = ax1 - ax0
    hA = ay1 - ay0
    cxA = ax0 + 0.5 * wA
    cyA = ay0 + 0.5 * hA
    dxv = deltas4[:, 0:1]
    dyv = deltas4[:, 1:2]
    dwv = deltas4[:, 2:3]
    dhv = deltas4[:, 3:4]
    pcx = dxv * wA + cxA
    pcy = dyv * hA + cyA
    pw = jnp.exp(dwv) * wA
    ph = jnp.exp(dhv) * hA
    pb0 = pcx - 0.5 * pw
    pb1 = pcy - 0.5 * ph
    pb2 = pcx + 0.5 * pw
    pb3 = pcy + 0.5 * ph
    area_p = (pb2 - pb0) * (pb3 - pb1)
    area_t = (tx1 - tx0) * (ty1 - ty0)
    iw = jnp.clip(jnp.minimum(pb2, tx1) - jnp.maximum(pb0, tx0), 0.0, None)
    ih = jnp.clip(jnp.minimum(pb3, ty1) - jnp.maximum(pb1, ty0), 0.0, None)
    inter = iw * ih
    union = area_p + area_t - inter
    iou = inter / (union + 1e-7)
    ew = jnp.maximum(pb2, tx1) - jnp.minimum(pb0, tx0)
    eh = jnp.maximum(pb3, ty1) - jnp.minimum(pb1, ty0)
    enc = ew * eh
    giou = iou - (enc - union) / (enc + 1e-7)
    gl_part = jnp.sum((1.0 - giou) * posf)

    ones = jnp.ones((1, 128), jnp.float32)
    upd = jnp.concatenate(
        [qfl_part * ones, npos_part * ones, dfl_part * ones, gl_part * ones], axis=0)
    acc_ref[...] = acc_ref[...] + upd


def kernel(cls_out, reg_out, anchors, gt_boxes, gt_labels):
    pad = jnp.tile(jnp.array([[1e8, 1e8, 1e8 + 8.0, 1e8 + 8.0]], jnp.float32),
                   (NP - N, 1))
    anchT = jnp.concatenate([anchors, pad], axis=0).T
    labf = gt_labels.astype(jnp.float32).reshape(B, M, 1)

    tgt = pl.pallas_call(
        _match_kernel,
        grid=(B,),
        in_specs=[
            pl.BlockSpec((4, NP), lambda b: (0, 0)),
            pl.BlockSpec((1, M, 4), lambda b: (b, 0, 0)),
            pl.BlockSpec((1, M, 1), lambda b: (b, 0, 0)),
        ],
        out_specs=pl.BlockSpec((1, 8, NP), lambda b: (b, 0, 0)),
        out_shape=jax.ShapeDtypeStruct((B, 8, NP), jnp.float32),
    )(anchT, gt_boxes, labf)

    tgtT = tgt[:, :, :N].transpose(0, 2, 1)

    acc = pl.pallas_call(
        _loss_kernel,
        grid=(B, NT),
        in_specs=[
            pl.BlockSpec((1, T, C), lambda b, n: (b, n, 0)),
            pl.BlockSpec((1, T, 4 * NB), lambda b, n: (b, n, 0)),
            pl.BlockSpec((T, 4), lambda b, n: (n, 0)),
            pl.BlockSpec((1, T, 8), lambda b, n: (b, n, 0)),
        ],
        out_specs=pl.BlockSpec((4, 128), lambda b, n: (0, 0)),
        out_shape=jax.ShapeDtypeStruct((4, 128), jnp.float32),
    )(cls_out, reg_out, anchors, tgtT)

    accs = acc
    qfl_sum = accs[0, 0]
    npos = accs[1, 0]
    dfl_sum = accs[2, 0]
    gl_sum = accs[3, 0]
    qfl = qfl_sum / jnp.maximum(npos, 1.0)
    dfl = dfl_sum / jnp.maximum(npos * 4.0, 1.0)
    gl = gl_sum / jnp.maximum(npos, 1.0)
    return qfl + dfl + GIOU_W * gl
